# Initial kernel scaffold; baseline (speedup 1.0000x reference)
import jax
import jax.numpy as jnp
from jax.experimental import pallas as pl


def kernel(x, state):
    raise NotImplementedError("write your pallas kernel here")


# speedup vs baseline: 6.0028x; 6.0028x over previous
import jax
import jax.numpy as jnp
from jax.experimental import pallas as pl
from jax.experimental.pallas import tpu as pltpu

BATCH = 512
NUM_NEURONS = 8192
HISTORY_LEN = 8

_BB = 8


def _push_body(x_ref, st_ref, out_ref):
    xb = x_ref[...]
    out_ref[...] = jnp.concatenate(
        [xb[:, None, :], st_ref[:, : HISTORY_LEN - 1, :]], axis=1
    )


def kernel(x, state):
    st = jnp.swapaxes(state, 1, 2)
    grid = (BATCH // _BB,)
    out_t = pl.pallas_call(
        _push_body,
        grid=grid,
        in_specs=[
            pl.BlockSpec((_BB, NUM_NEURONS), lambda b: (b, 0)),
            pl.BlockSpec((_BB, HISTORY_LEN, NUM_NEURONS), lambda b: (b, 0, 0)),
        ],
        out_specs=pl.BlockSpec(
            (_BB, HISTORY_LEN, NUM_NEURONS), lambda b: (b, 0, 0)
        ),
        out_shape=jax.ShapeDtypeStruct((BATCH, HISTORY_LEN, NUM_NEURONS), jnp.float32),
    )(x, st)
    return jnp.swapaxes(out_t, 1, 2)


# speedup vs baseline: 6.6244x; 1.1035x over previous
import jax
import jax.numpy as jnp
from jax.experimental import pallas as pl

BATCH = 512
NUM_NEURONS = 8192
HISTORY_LEN = 8

_BB = 32


def _push_body(x_ref, st_ref, out_ref):
    xb = x_ref[...]
    out_ref[...] = jnp.concatenate(
        [xb[:, None, :], st_ref[:, : HISTORY_LEN - 1, :]], axis=1
    )


def kernel(x, state):
    st = jnp.swapaxes(state, 1, 2)
    out_t = pl.pallas_call(
        _push_body,
        grid=(BATCH // _BB,),
        in_specs=[
            pl.BlockSpec((_BB, NUM_NEURONS), lambda b: (b, 0)),
            pl.BlockSpec((_BB, HISTORY_LEN, NUM_NEURONS), lambda b: (b, 0, 0)),
        ],
        out_specs=pl.BlockSpec(
            (_BB, HISTORY_LEN, NUM_NEURONS), lambda b: (b, 0, 0)
        ),
        out_shape=jax.ShapeDtypeStruct((BATCH, HISTORY_LEN, NUM_NEURONS), jnp.float32),
    )(x, st)
    return jnp.swapaxes(out_t, 1, 2)
